# baseline (speedup 1.0000x reference)
import functools

import jax
import jax.numpy as jnp
from jax import lax
from jax.experimental import pallas as pl
from jax.experimental.pallas import tpu as pltpu
from jax.experimental.pallas import tpu_sc as plsc

T = 128
H = 1024
E = 64
I = 512
EPB = 2
NW = 32
TPW = T // NW


def _logits_body(x_ref, wg_ref, o_ref):
    o_ref[...] = jnp.dot(x_ref[...], wg_ref[...],
                         preferred_element_type=jnp.float32)


def _sc_router_body(logits_hbm, bias_hbm, out_hbm, lg_v, bias_v, cmb_v):
    wid = lax.axis_index("s") * 2 + lax.axis_index("c")
    base = wid * TPW
    pltpu.sync_copy(logits_hbm.at[pl.ds(base, TPW)], lg_v)
    pltpu.sync_copy(bias_hbm, bias_v)
    iota = lax.iota(jnp.int32, 16).astype(jnp.float32)
    for t in range(TPW):
        s_j = []
        choice_j = []
        idx_j = []
        for j in range(E // 16):
            lg = lg_v[t, pl.ds(16 * j, 16)]
            s = 1.0 / (1.0 + jnp.exp(-lg))
            s_j.append(s)
            choice_j.append(s + bias_v[pl.ds(16 * j, 16)])
            idx_j.append(iota + (16.0 * j))
        m1 = jnp.max(choice_j[0])
        for j in range(1, E // 16):
            m1 = jnp.maximum(m1, jnp.max(choice_j[j]))
        idx1 = jnp.min(jnp.where(choice_j[0] == m1, idx_j[0], 1e9))
        for j in range(1, E // 16):
            idx1 = jnp.minimum(
                idx1, jnp.min(jnp.where(choice_j[j] == m1, idx_j[j], 1e9)))
        w1 = sum(jnp.sum(jnp.where(idx_j[j] == idx1, s_j[j], 0.0))
                 for j in range(E // 16))
        ch2_j = [jnp.where(idx_j[j] == idx1, -1e30, choice_j[j])
                 for j in range(E // 16)]
        m2 = jnp.max(ch2_j[0])
        for j in range(1, E // 16):
            m2 = jnp.maximum(m2, jnp.max(ch2_j[j]))
        idx2 = jnp.min(jnp.where(ch2_j[0] == m2, idx_j[0], 1e9))
        for j in range(1, E // 16):
            idx2 = jnp.minimum(
                idx2, jnp.min(jnp.where(ch2_j[j] == m2, idx_j[j], 1e9)))
        w2 = sum(jnp.sum(jnp.where(idx_j[j] == idx2, s_j[j], 0.0))
                 for j in range(E // 16))
        den = w1 + w2
        for j in range(E // 16):
            num = (jnp.where(idx_j[j] == idx1, w1, 0.0) +
                   jnp.where(idx_j[j] == idx2, w2, 0.0))
            cmb_v[t, pl.ds(16 * j, 16)] = num / jnp.full((16,), den,
                                                         jnp.float32)
    pltpu.sync_copy(cmb_v, out_hbm.at[pl.ds(base, TPW)])


def _sc_router(logits, bias1d):
    mesh = plsc.VectorSubcoreMesh(core_axis_name="c", subcore_axis_name="s")
    k = functools.partial(
        pl.kernel, mesh=mesh,
        out_type=jax.ShapeDtypeStruct((T, E), jnp.float32),
        scratch_types=[
            pltpu.VMEM((TPW, E), jnp.float32),
            pltpu.VMEM((E,), jnp.float32),
            pltpu.VMEM((TPW, E), jnp.float32),
        ],
        compiler_params=pltpu.CompilerParams(needs_layout_passes=False),
    )(_sc_router_body)
    return k(logits, bias1d)


def _moe_body(x_ref, combine_ref, wgu_g_ref, wgu_u_ref, wd_a_ref,
              wd_b_ref, wsgu_ref, wsd_ref, o_ref):
    e = pl.program_id(0)
    x = x_ref[...]

    @pl.when(e == 0)
    def _shared():
        gu = jnp.dot(x, wsgu_ref[...], preferred_element_type=jnp.float32)
        act = jax.nn.silu(gu[:, :I]) * gu[:, I:]
        o_ref[...] = jnp.dot(act, wsd_ref[...], preferred_element_type=jnp.float32)

    xb = x.astype(jnp.bfloat16)
    cols = jax.lax.broadcasted_iota(jnp.int32, (T, E), 1)
    acc = jnp.zeros((T, H), jnp.float32)
    for j in range(EPB):
        gate = jnp.dot(xb, wgu_g_ref[j].astype(jnp.bfloat16),
                       preferred_element_type=jnp.float32)
        up = jnp.dot(xb, wgu_u_ref[j].astype(jnp.bfloat16),
                     preferred_element_type=jnp.float32)
        act = (jax.nn.silu(gate) * up).astype(jnp.bfloat16)
        oe = (jnp.dot(act[:, :I // 2], wd_a_ref[j].astype(jnp.bfloat16),
                      preferred_element_type=jnp.float32) +
              jnp.dot(act[:, I // 2:], wd_b_ref[j].astype(jnp.bfloat16),
                      preferred_element_type=jnp.float32))
        w_e = jnp.sum(jnp.where(cols == e * EPB + j, combine_ref[...], 0.0),
                      axis=1, keepdims=True)
        acc += w_e * oe
    o_ref[...] += acc


def kernel(hidden_states, Wg, Wgu, Wd, Ws_gu, Ws_d, expert_bias):
    logits = pl.pallas_call(
        _logits_body,
        out_shape=jax.ShapeDtypeStruct((T, E), jnp.float32),
    )(hidden_states, Wg)
    combine = _sc_router(logits, expert_bias)
    return pl.pallas_call(
        _moe_body,
        grid=(E // EPB,),
        in_specs=[
            pl.BlockSpec((T, H), lambda e: (0, 0)),
            pl.BlockSpec((T, E), lambda e: (0, 0)),
            pl.BlockSpec((EPB, H, I), lambda e: (e, 0, 0)),
            pl.BlockSpec((EPB, H, I), lambda e: (e, 0, 1)),
            pl.BlockSpec((EPB, I // 2, H), lambda e: (e, 0, 0)),
            pl.BlockSpec((EPB, I // 2, H), lambda e: (e, 1, 0)),
            pl.BlockSpec((H, 2 * I), lambda e: (0, 0)),
            pl.BlockSpec((I, H), lambda e: (0, 0)),
        ],
        out_specs=pl.BlockSpec((T, H), lambda e: (0, 0)),
        out_shape=jax.ShapeDtypeStruct((T, H), jnp.float32),
    )(hidden_states, combine, Wgu, Wgu, Wd, Wd, Ws_gu, Ws_d)


# speedup vs baseline: 1.1591x; 1.1591x over previous
import jax
import jax.numpy as jnp
from jax.experimental import pallas as pl
from jax.experimental.pallas import tpu as pltpu

T = 128
H = 1024
E = 64
I = 512
EPB = 2


def _moe_body(x_ref, wg_ref, bias_ref, wgu_g_ref, wgu_u_ref, wd_a_ref,
              wd_b_ref, wsgu_ref, wsd_ref, o_ref, combine_ref):
    e = pl.program_id(0)
    x = x_ref[...]

    @pl.when(e == 0)
    def _router_and_shared():
        logits = jnp.dot(x, wg_ref[...], preferred_element_type=jnp.float32)
        s = jax.nn.sigmoid(logits)
        choice = s + bias_ref[...]
        cols = jax.lax.broadcasted_iota(jnp.int32, (T, E), 1)
        idx1 = jnp.argmax(choice, axis=1)
        m1 = cols == idx1[:, None]
        choice2 = jnp.where(m1, -jnp.inf, choice)
        idx2 = jnp.argmax(choice2, axis=1)
        m2 = cols == idx2[:, None]
        w1 = jnp.sum(jnp.where(m1, s, 0.0), axis=1)
        w2 = jnp.sum(jnp.where(m2, s, 0.0), axis=1)
        inv = 1.0 / (w1 + w2)
        combine_ref[...] = (jnp.where(m1, (w1 * inv)[:, None], 0.0) +
                            jnp.where(m2, (w2 * inv)[:, None], 0.0))
        gu = jnp.dot(x, wsgu_ref[...], preferred_element_type=jnp.float32)
        act = jax.nn.silu(gu[:, :I]) * gu[:, I:]
        o_ref[...] = jnp.dot(act, wsd_ref[...], preferred_element_type=jnp.float32)

    xb = x.astype(jnp.bfloat16)
    cols = jax.lax.broadcasted_iota(jnp.int32, (T, E), 1)
    acc = jnp.zeros((T, H), jnp.float32)
    for j in range(EPB):
        gate = jnp.dot(xb, wgu_g_ref[j].astype(jnp.bfloat16),
                       preferred_element_type=jnp.float32)
        up = jnp.dot(xb, wgu_u_ref[j].astype(jnp.bfloat16),
                     preferred_element_type=jnp.float32)
        act = (jax.nn.silu(gate) * up).astype(jnp.bfloat16)
        oe = (jnp.dot(act[:, :I // 2], wd_a_ref[j].astype(jnp.bfloat16),
                      preferred_element_type=jnp.float32) +
              jnp.dot(act[:, I // 2:], wd_b_ref[j].astype(jnp.bfloat16),
                      preferred_element_type=jnp.float32))
        w_e = jnp.sum(jnp.where(cols == e * EPB + j, combine_ref[...], 0.0),
                      axis=1, keepdims=True)
        acc += w_e * oe
    o_ref[...] += acc


def kernel(hidden_states, Wg, Wgu, Wd, Ws_gu, Ws_d, expert_bias):
    bias2d = expert_bias.reshape(1, E)
    return pl.pallas_call(
        _moe_body,
        grid=(E // EPB,),
        in_specs=[
            pl.BlockSpec((T, H), lambda e: (0, 0)),
            pl.BlockSpec((H, E), lambda e: (0, 0)),
            pl.BlockSpec((1, E), lambda e: (0, 0)),
            pl.BlockSpec((EPB, H, I), lambda e: (e, 0, 0)),
            pl.BlockSpec((EPB, H, I), lambda e: (e, 0, 1)),
            pl.BlockSpec((EPB, I // 2, H), lambda e: (e, 0, 0)),
            pl.BlockSpec((EPB, I // 2, H), lambda e: (e, 1, 0)),
            pl.BlockSpec((H, 2 * I), lambda e: (0, 0)),
            pl.BlockSpec((I, H), lambda e: (0, 0)),
        ],
        out_specs=pl.BlockSpec((T, H), lambda e: (0, 0)),
        out_shape=jax.ShapeDtypeStruct((T, H), jnp.float32),
        scratch_shapes=[pltpu.VMEM((T, E), jnp.float32)],
    )(hidden_states, Wg, bias2d, Wgu, Wgu, Wd, Wd, Ws_gu, Ws_d)


# speedup vs baseline: 1.1716x; 1.0107x over previous
import jax
import jax.numpy as jnp
from jax.experimental import pallas as pl
from jax.experimental.pallas import tpu as pltpu

T = 128
H = 1024
E = 64
I = 512
EPB = 2


def _moe_body(x_ref, wg_ref, bias_ref, wgu_g_ref, wgu_u_ref, wd_a_ref,
              wd_b_ref, wd_c_ref, wd_d_ref, wsgu_ref, wsd_ref, o_ref,
              combine_ref):
    e = pl.program_id(0)
    x = x_ref[...]

    @pl.when(e == 0)
    def _router_and_shared():
        logits = jnp.dot(x, wg_ref[...], preferred_element_type=jnp.float32)
        s = jax.nn.sigmoid(logits)
        choice = s + bias_ref[...]
        cols = jax.lax.broadcasted_iota(jnp.int32, (T, E), 1)
        idx1 = jnp.argmax(choice, axis=1)
        m1 = cols == idx1[:, None]
        choice2 = jnp.where(m1, -jnp.inf, choice)
        idx2 = jnp.argmax(choice2, axis=1)
        m2 = cols == idx2[:, None]
        w1 = jnp.sum(jnp.where(m1, s, 0.0), axis=1)
        w2 = jnp.sum(jnp.where(m2, s, 0.0), axis=1)
        inv = 1.0 / (w1 + w2)
        combine_ref[...] = (jnp.where(m1, (w1 * inv)[:, None], 0.0) +
                            jnp.where(m2, (w2 * inv)[:, None], 0.0))
        gu = jnp.dot(x, wsgu_ref[...], preferred_element_type=jnp.float32)
        act = jax.nn.silu(gu[:, :I]) * gu[:, I:]
        o_ref[...] = jnp.dot(act, wsd_ref[...], preferred_element_type=jnp.float32)

    xb = x.astype(jnp.bfloat16)
    cols = jax.lax.broadcasted_iota(jnp.int32, (T, E), 1)
    acc = jnp.zeros((T, H), jnp.float32)
    for j in range(EPB):
        gate = jnp.dot(xb, wgu_g_ref[j].astype(jnp.bfloat16),
                       preferred_element_type=jnp.float32)
        up = jnp.dot(xb, wgu_u_ref[j].astype(jnp.bfloat16),
                     preferred_element_type=jnp.float32)
        act = (jax.nn.silu(gate) * up).astype(jnp.bfloat16)
        oe = (jnp.dot(act[:, :I // 4], wd_a_ref[j].astype(jnp.bfloat16),
                      preferred_element_type=jnp.float32) +
              jnp.dot(act[:, I // 4:I // 2], wd_b_ref[j].astype(jnp.bfloat16),
                      preferred_element_type=jnp.float32) +
              jnp.dot(act[:, I // 2:3 * I // 4], wd_c_ref[j].astype(jnp.bfloat16),
                      preferred_element_type=jnp.float32) +
              jnp.dot(act[:, 3 * I // 4:], wd_d_ref[j].astype(jnp.bfloat16),
                      preferred_element_type=jnp.float32))
        w_e = jnp.sum(jnp.where(cols == e * EPB + j, combine_ref[...], 0.0),
                      axis=1, keepdims=True)
        acc += w_e * oe
    o_ref[...] += acc


def kernel(hidden_states, Wg, Wgu, Wd, Ws_gu, Ws_d, expert_bias):
    bias2d = expert_bias.reshape(1, E)
    return pl.pallas_call(
        _moe_body,
        grid=(E // EPB,),
        in_specs=[
            pl.BlockSpec((T, H), lambda e: (0, 0)),
            pl.BlockSpec((H, E), lambda e: (0, 0)),
            pl.BlockSpec((1, E), lambda e: (0, 0)),
            pl.BlockSpec((EPB, H, I), lambda e: (e, 0, 0)),
            pl.BlockSpec((EPB, H, I), lambda e: (e, 0, 1)),
            pl.BlockSpec((EPB, I // 4, H), lambda e: (e, 0, 0)),
            pl.BlockSpec((EPB, I // 4, H), lambda e: (e, 1, 0)),
            pl.BlockSpec((EPB, I // 4, H), lambda e: (e, 2, 0)),
            pl.BlockSpec((EPB, I // 4, H), lambda e: (e, 3, 0)),
            pl.BlockSpec((H, 2 * I), lambda e: (0, 0)),
            pl.BlockSpec((I, H), lambda e: (0, 0)),
        ],
        out_specs=pl.BlockSpec((T, H), lambda e: (0, 0)),
        out_shape=jax.ShapeDtypeStruct((T, H), jnp.float32),
        scratch_shapes=[pltpu.VMEM((T, E), jnp.float32)],
    )(hidden_states, Wg, bias2d, Wgu, Wgu, Wd, Wd, Wd, Wd, Ws_gu, Ws_d)


# speedup vs baseline: 1.1733x; 1.0015x over previous
import jax
import jax.numpy as jnp
from jax.experimental import pallas as pl
from jax.experimental.pallas import tpu as pltpu

T = 128
H = 1024
E = 64
I = 512
EPB = 2


def _moe_body(x_ref, wg_ref, bias_ref, wgu_a_ref, wgu_b_ref, wgu_c_ref,
              wgu_d_ref, wd_a_ref, wd_b_ref, wsgu_ref, wsd_ref, o_ref,
              combine_ref):
    e = pl.program_id(0)
    x = x_ref[...]

    @pl.when(e == 0)
    def _router_and_shared():
        logits = jnp.dot(x, wg_ref[...], preferred_element_type=jnp.float32)
        s = jax.nn.sigmoid(logits)
        choice = s + bias_ref[...]
        cols = jax.lax.broadcasted_iota(jnp.int32, (T, E), 1)
        idx1 = jnp.argmax(choice, axis=1)
        m1 = cols == idx1[:, None]
        choice2 = jnp.where(m1, -jnp.inf, choice)
        idx2 = jnp.argmax(choice2, axis=1)
        m2 = cols == idx2[:, None]
        w1 = jnp.sum(jnp.where(m1, s, 0.0), axis=1)
        w2 = jnp.sum(jnp.where(m2, s, 0.0), axis=1)
        inv = 1.0 / (w1 + w2)
        combine_ref[...] = (jnp.where(m1, (w1 * inv)[:, None], 0.0) +
                            jnp.where(m2, (w2 * inv)[:, None], 0.0))
        gu = jnp.dot(x, wsgu_ref[...], preferred_element_type=jnp.float32)
        act = jax.nn.silu(gu[:, :I]) * gu[:, I:]
        o_ref[...] = jnp.dot(act, wsd_ref[...], preferred_element_type=jnp.float32)

    xb = x.astype(jnp.bfloat16)
    cols = jax.lax.broadcasted_iota(jnp.int32, (T, E), 1)
    acc = jnp.zeros((T, H), jnp.float32)
    for j in range(EPB):
        gate_a = jnp.dot(xb, wgu_a_ref[j].astype(jnp.bfloat16),
                         preferred_element_type=jnp.float32)
        gate_b = jnp.dot(xb, wgu_b_ref[j].astype(jnp.bfloat16),
                         preferred_element_type=jnp.float32)
        up_a = jnp.dot(xb, wgu_c_ref[j].astype(jnp.bfloat16),
                       preferred_element_type=jnp.float32)
        up_b = jnp.dot(xb, wgu_d_ref[j].astype(jnp.bfloat16),
                       preferred_element_type=jnp.float32)
        act_a = (jax.nn.silu(gate_a) * up_a).astype(jnp.bfloat16)
        act_b = (jax.nn.silu(gate_b) * up_b).astype(jnp.bfloat16)
        oe = (jnp.dot(act_a, wd_a_ref[j].astype(jnp.bfloat16),
                      preferred_element_type=jnp.float32) +
              jnp.dot(act_b, wd_b_ref[j].astype(jnp.bfloat16),
                      preferred_element_type=jnp.float32))
        w_e = jnp.sum(jnp.where(cols == e * EPB + j, combine_ref[...], 0.0),
                      axis=1, keepdims=True)
        acc += w_e * oe
    o_ref[...] += acc


def kernel(hidden_states, Wg, Wgu, Wd, Ws_gu, Ws_d, expert_bias):
    bias2d = expert_bias.reshape(1, E)
    return pl.pallas_call(
        _moe_body,
        grid=(E // EPB,),
        in_specs=[
            pl.BlockSpec((T, H), lambda e: (0, 0)),
            pl.BlockSpec((H, E), lambda e: (0, 0)),
            pl.BlockSpec((1, E), lambda e: (0, 0)),
            pl.BlockSpec((EPB, H, I // 2), lambda e: (e, 0, 0)),
            pl.BlockSpec((EPB, H, I // 2), lambda e: (e, 0, 1)),
            pl.BlockSpec((EPB, H, I // 2), lambda e: (e, 0, 2)),
            pl.BlockSpec((EPB, H, I // 2), lambda e: (e, 0, 3)),
            pl.BlockSpec((EPB, I // 2, H), lambda e: (e, 0, 0)),
            pl.BlockSpec((EPB, I // 2, H), lambda e: (e, 1, 0)),
            pl.BlockSpec((H, 2 * I), lambda e: (0, 0)),
            pl.BlockSpec((I, H), lambda e: (0, 0)),
        ],
        out_specs=pl.BlockSpec((T, H), lambda e: (0, 0)),
        out_shape=jax.ShapeDtypeStruct((T, H), jnp.float32),
        scratch_shapes=[pltpu.VMEM((T, E), jnp.float32)],
    )(hidden_states, Wg, bias2d, Wgu, Wgu, Wgu, Wgu, Wd, Wd, Ws_gu, Ws_d)
